# baseline (speedup 1.0000x reference)
import jax
import jax.numpy as jnp
import numpy as np
from jax.experimental import pallas as pl
from jax.experimental.pallas import tpu as pltpu

PROJ_DIM = 128
CLASS_NUM = 7
QUEUE_SIZE = 150
NQ = CLASS_NUM * QUEUE_SIZE
TEMPERATURE = 0.5
LN2 = float(np.log(2.0))
C_SCALE = (TEMPERATURE * LN2) ** 2
BLK = 4096
CPAD = 8


def _colsum_t(x_bf):
    ones = jnp.ones((1, x_bf.shape[1]), dtype=jnp.bfloat16)
    return jax.lax.dot_general(
        ones, x_bf, (((1,), (1,)), ((), ())), preferred_element_type=jnp.float32
    )


def _pgc_kernel(act_ref, ema_ref, lab_ref, qbf_ref, seg_ref, out_ref):
    abf = act_ref[...]
    ebf = ema_ref[...]
    pt = lab_ref[...]
    qbf = qbf_ref[...]

    ssa_t = _colsum_t(abf * abf)
    sse_t = _colsum_t(ebf * ebf)
    dae_t = _colsum_t(abf * ebf)

    inv_at = jax.lax.rsqrt(jnp.maximum(ssa_t * C_SCALE, 1e-24 * C_SCALE))
    inv_e = jax.lax.rsqrt(jnp.maximum(sse_t, 1e-24))
    spos = dae_t * inv_at * inv_e

    s_raw = jax.lax.dot_general(
        qbf, abf, (((1,), (1,)), ((), ())), preferred_element_type=jnp.float32
    )
    sims = jnp.exp2(s_raw * inv_at)
    total = jnp.sum(sims, axis=0, keepdims=True)

    colmax = jnp.max(pt, axis=0, keepdims=True)
    cls_idx = jax.lax.broadcasted_iota(jnp.int32, pt.shape, 0)
    label = jnp.min(
        jnp.where(pt == colmax, cls_idx, CLASS_NUM), axis=0, keepdims=True
    )
    onehot = (
        jax.lax.broadcasted_iota(jnp.int32, (CPAD, pt.shape[1]), 0) == label
    ).astype(jnp.float32)

    qsum_bf = jax.lax.dot_general(
        seg_ref[...], qbf, (((0,), (0,)), ((), ())),
        preferred_element_type=jnp.float32,
    ).astype(jnp.bfloat16)
    h = jax.lax.dot_general(
        qsum_bf, abf, (((1,), (1,)), ((), ())), preferred_element_type=jnp.float32
    )
    pos = jnp.sum(onehot * h, axis=0, keepdims=True) * inv_at

    denom = jnp.exp2(spos) + total
    partial = (
        (QUEUE_SIZE + 1) * jnp.sum(jnp.log2(denom)) - jnp.sum(spos) - jnp.sum(pos)
    ) * (LN2 / (QUEUE_SIZE + 1))

    @pl.when(pl.program_id(0) == 0)
    def _init():
        out_ref[0, 0] = 0.0

    out_ref[0, 0] += partial


def kernel(activation, ema_activation, pseudo_label, queue_list):
    batch = activation.shape[0]
    grid = batch // BLK
    seg_np = np.zeros((NQ, CPAD), dtype=np.float32)
    for c in range(CLASS_NUM):
        seg_np[c * QUEUE_SIZE : (c + 1) * QUEUE_SIZE, c] = 1.0
    seg = jnp.asarray(seg_np, dtype=jnp.bfloat16)
    a_bf = activation.astype(jnp.bfloat16)
    e_bf = ema_activation.astype(jnp.bfloat16)
    q_bf = queue_list.astype(jnp.bfloat16)
    pl_t = pseudo_label.T
    out = pl.pallas_call(
        _pgc_kernel,
        grid=(grid,),
        in_specs=[
            pl.BlockSpec((BLK, PROJ_DIM), lambda i: (i, 0)),
            pl.BlockSpec((BLK, PROJ_DIM), lambda i: (i, 0)),
            pl.BlockSpec((CLASS_NUM, BLK), lambda i: (0, i)),
            pl.BlockSpec((NQ, PROJ_DIM), lambda i: (0, 0)),
            pl.BlockSpec((NQ, CPAD), lambda i: (0, 0)),
        ],
        out_specs=pl.BlockSpec((1, 1), lambda i: (0, 0), memory_space=pltpu.SMEM),
        out_shape=jax.ShapeDtypeStruct((1, 1), jnp.float32),
    )(a_bf, e_bf, pl_t, q_bf, seg)
    return out[0, 0] / batch


# speedup vs baseline: 1.4315x; 1.4315x over previous
import jax
import jax.numpy as jnp
import numpy as np
from jax.experimental import pallas as pl
from jax.experimental.pallas import tpu as pltpu

PROJ_DIM = 128
CLASS_NUM = 7
QUEUE_SIZE = 150
NQ = CLASS_NUM * QUEUE_SIZE
TEMPERATURE = 0.5
LN2 = float(np.log(2.0))
C_SCALE = (TEMPERATURE * LN2) ** 2
BLK = 4096
CPAD = 8


def _colsum_t(x_bf):
    ones = jnp.ones((1, x_bf.shape[1]), dtype=jnp.bfloat16)
    return jax.lax.dot_general(
        ones, x_bf, (((1,), (1,)), ((), ())), preferred_element_type=jnp.float32
    )


def _pgc_kernel(act_ref, ema_ref, lab_ref, q_ref, seg_ref, out_ref):
    abf = act_ref[...].astype(jnp.bfloat16)
    ebf = ema_ref[...].astype(jnp.bfloat16)
    pt = lab_ref[...]
    qbf = q_ref[...].astype(jnp.bfloat16)

    ssa_t = _colsum_t(abf * abf)
    sse_t = _colsum_t(ebf * ebf)
    dae_t = _colsum_t(abf * ebf)

    inv_at = jax.lax.rsqrt(jnp.maximum(ssa_t * C_SCALE, 1e-24 * C_SCALE))
    inv_e = jax.lax.rsqrt(jnp.maximum(sse_t, 1e-24))
    spos = dae_t * inv_at * inv_e

    s_raw = jax.lax.dot_general(
        qbf, abf, (((1,), (1,)), ((), ())), preferred_element_type=jnp.float32
    )
    sims = jnp.exp2(s_raw * inv_at)
    total = jnp.sum(sims, axis=0, keepdims=True)

    colmax = jnp.max(pt, axis=0, keepdims=True)
    cls_idx = jax.lax.broadcasted_iota(jnp.int32, pt.shape, 0)
    label = jnp.min(
        jnp.where(pt == colmax, cls_idx, CLASS_NUM), axis=0, keepdims=True
    )
    onehot = (
        jax.lax.broadcasted_iota(jnp.int32, (CPAD, pt.shape[1]), 0) == label
    ).astype(jnp.float32)

    qsum_bf = jax.lax.dot_general(
        seg_ref[...], qbf, (((0,), (0,)), ((), ())),
        preferred_element_type=jnp.float32,
    ).astype(jnp.bfloat16)
    h = jax.lax.dot_general(
        qsum_bf, abf, (((1,), (1,)), ((), ())), preferred_element_type=jnp.float32
    )
    pos = jnp.sum(onehot * h, axis=0, keepdims=True) * inv_at

    denom = jnp.exp2(spos) + total
    partial = (
        (QUEUE_SIZE + 1) * jnp.sum(jnp.log2(denom)) - jnp.sum(spos) - jnp.sum(pos)
    ) * (LN2 / (QUEUE_SIZE + 1))

    @pl.when(pl.program_id(0) == 0)
    def _init():
        out_ref[0, 0] = 0.0

    out_ref[0, 0] += partial


def kernel(activation, ema_activation, pseudo_label, queue_list):
    batch = activation.shape[0]
    grid = batch // BLK
    seg_np = np.zeros((NQ, CPAD), dtype=np.float32)
    for c in range(CLASS_NUM):
        seg_np[c * QUEUE_SIZE : (c + 1) * QUEUE_SIZE, c] = 1.0
    seg = jnp.asarray(seg_np, dtype=jnp.bfloat16)
    pl_t = pseudo_label.T
    out = pl.pallas_call(
        _pgc_kernel,
        grid=(grid,),
        in_specs=[
            pl.BlockSpec((BLK, PROJ_DIM), lambda i: (i, 0)),
            pl.BlockSpec((BLK, PROJ_DIM), lambda i: (i, 0)),
            pl.BlockSpec((CLASS_NUM, BLK), lambda i: (0, i)),
            pl.BlockSpec((NQ, PROJ_DIM), lambda i: (0, 0)),
            pl.BlockSpec((NQ, CPAD), lambda i: (0, 0)),
        ],
        out_specs=pl.BlockSpec((1, 1), lambda i: (0, 0), memory_space=pltpu.SMEM),
        out_shape=jax.ShapeDtypeStruct((1, 1), jnp.float32),
    )(activation, ema_activation, pl_t, queue_list, seg)
    return out[0, 0] / batch
